# baseline (device time: 534688 ns/iter reference)
import jax
import jax.numpy as jnp
from jax import lax
from jax.experimental import pallas as pl
from jax.experimental.pallas import tpu as pltpu

N_DEV = 16


def kernel(A, B):
    m, k = A.shape
    _, n = B.shape
    chunk = m // N_DEV

    def body(a_ref, b_ref, out_ref, comm_ref, send_sems, recv_sems, credit_sem):
        my = lax.axis_index("i")
        left = jnp.mod(my + N_DEV - 1, N_DEV)
        right = jnp.mod(my + 1, N_DEV)

        def partial_chunk(c):
            a_blk = a_ref[pl.ds(c * chunk, chunk), :]
            return jnp.dot(a_blk, b_ref[...], preferred_element_type=jnp.float32)

        barrier_sem = pltpu.get_barrier_semaphore()
        for nbr in (left, right):
            pl.semaphore_signal(
                barrier_sem, inc=1,
                device_id=(nbr,), device_id_type=pl.DeviceIdType.MESH,
            )
        pl.semaphore_wait(barrier_sem, 2)

        comm_ref[0, :, :] = partial_chunk(my)

        pl.semaphore_signal(
            credit_sem, inc=1,
            device_id=(left,), device_id_type=pl.DeviceIdType.MESH,
        )

        n_steps = 2 * N_DEV - 2
        for g in range(n_steps):
            send_slot = g % 2
            recv_slot = (g + 1) % 2

            pl.semaphore_wait(credit_sem, 1)

            rdma = pltpu.make_async_remote_copy(
                src_ref=comm_ref.at[send_slot],
                dst_ref=comm_ref.at[recv_slot],
                send_sem=send_sems.at[send_slot],
                recv_sem=recv_sems.at[recv_slot],
                device_id=(right,),
                device_id_type=pl.DeviceIdType.MESH,
            )
            rdma.start()
            rdma.wait()

            if g < N_DEV - 2:
                c = jnp.mod(my - (g + 1), N_DEV)
                comm_ref[recv_slot, :, :] = (
                    comm_ref[recv_slot, :, :] + partial_chunk(c)
                )
            elif g == N_DEV - 2:
                c = jnp.mod(my + 1, N_DEV)
                z = comm_ref[recv_slot, :, :] + partial_chunk(c)
                z = z / (1.0 + jnp.exp(-z))
                comm_ref[recv_slot, :, :] = z
                out_ref[pl.ds(c * chunk, chunk), :] = z
            else:
                t = g - (N_DEV - 1)
                c = jnp.mod(my - t, N_DEV)
                out_ref[pl.ds(c * chunk, chunk), :] = comm_ref[recv_slot, :, :]

            if g < n_steps - 1:
                pl.semaphore_signal(
                    credit_sem, inc=1,
                    device_id=(left,), device_id_type=pl.DeviceIdType.MESH,
                )

    return pl.pallas_call(
        body,
        out_shape=jax.ShapeDtypeStruct((m, n), jnp.float32),
        in_specs=[
            pl.BlockSpec(memory_space=pltpu.VMEM),
            pl.BlockSpec(memory_space=pltpu.VMEM),
        ],
        out_specs=pl.BlockSpec(memory_space=pltpu.VMEM),
        scratch_shapes=[
            pltpu.VMEM((2, chunk, n), jnp.float32),
            pltpu.SemaphoreType.DMA((2,)),
            pltpu.SemaphoreType.DMA((2,)),
            pltpu.SemaphoreType.REGULAR,
        ],
        compiler_params=pltpu.CompilerParams(collective_id=0),
    )(A, B)


# device time: 358707 ns/iter; 1.4906x vs baseline; 1.4906x over previous
import jax
import jax.numpy as jnp
from jax import lax
from jax.experimental import pallas as pl
from jax.experimental.pallas import tpu as pltpu

N_DEV = 16


def kernel(A, B):
    m, k = A.shape
    _, n = B.shape
    chunk = m // N_DEV
    half = chunk // 2

    def body(a_ref, b_ref, out_ref,
             comm_r, comm_l, send_r, recv_r, send_l, recv_l,
             credit_r, credit_l):
        my = lax.axis_index("i")
        left = jnp.mod(my + N_DEV - 1, N_DEV)
        right = jnp.mod(my + 1, N_DEV)

        def partial_top(c):
            a_blk = a_ref[pl.ds(c * chunk, half), :]
            return jnp.dot(a_blk, b_ref[...], preferred_element_type=jnp.float32)

        def partial_bot(c):
            a_blk = a_ref[pl.ds(c * chunk + half, half), :]
            return jnp.dot(a_blk, b_ref[...], preferred_element_type=jnp.float32)

        barrier_sem = pltpu.get_barrier_semaphore()
        for nbr in (left, right):
            pl.semaphore_signal(
                barrier_sem, inc=1,
                device_id=(nbr,), device_id_type=pl.DeviceIdType.MESH,
            )
        pl.semaphore_wait(barrier_sem, 2)

        comm_r[0, :, :] = partial_top(my)
        comm_l[0, :, :] = partial_bot(my)

        pl.semaphore_signal(credit_r, inc=1, device_id=(left,),
                            device_id_type=pl.DeviceIdType.MESH)
        pl.semaphore_signal(credit_l, inc=1, device_id=(right,),
                            device_id_type=pl.DeviceIdType.MESH)

        n_steps = 2 * N_DEV - 2
        for g in range(n_steps):
            s_slot = g % 2
            r_slot = (g + 1) % 2

            pl.semaphore_wait(credit_r, 1)
            pl.semaphore_wait(credit_l, 1)

            rdma_r = pltpu.make_async_remote_copy(
                src_ref=comm_r.at[s_slot], dst_ref=comm_r.at[r_slot],
                send_sem=send_r.at[s_slot], recv_sem=recv_r.at[r_slot],
                device_id=(right,), device_id_type=pl.DeviceIdType.MESH,
            )
            rdma_l = pltpu.make_async_remote_copy(
                src_ref=comm_l.at[s_slot], dst_ref=comm_l.at[r_slot],
                send_sem=send_l.at[s_slot], recv_sem=recv_l.at[r_slot],
                device_id=(left,), device_id_type=pl.DeviceIdType.MESH,
            )
            rdma_r.start()
            rdma_l.start()

            if g < N_DEV - 1:
                c_r = jnp.mod(my - (g + 1), N_DEV)
                c_l = jnp.mod(my + (g + 1), N_DEV)
                p_r = partial_top(c_r)
                p_l = partial_bot(c_l)
                rdma_r.wait()
                rdma_l.wait()
                if g < N_DEV - 2:
                    comm_r[r_slot, :, :] = comm_r[r_slot, :, :] + p_r
                    comm_l[r_slot, :, :] = comm_l[r_slot, :, :] + p_l
                else:
                    z_r = comm_r[r_slot, :, :] + p_r
                    z_r = z_r / (1.0 + jnp.exp(-z_r))
                    comm_r[r_slot, :, :] = z_r
                    out_ref[pl.ds(c_r * chunk, half), :] = z_r
                    z_l = comm_l[r_slot, :, :] + p_l
                    z_l = z_l / (1.0 + jnp.exp(-z_l))
                    comm_l[r_slot, :, :] = z_l
                    out_ref[pl.ds(c_l * chunk + half, half), :] = z_l
            else:
                t = g - (N_DEV - 1)
                c_r = jnp.mod(my - t, N_DEV)
                c_l = jnp.mod(my + t, N_DEV)
                rdma_r.wait()
                rdma_l.wait()
                out_ref[pl.ds(c_r * chunk, half), :] = comm_r[r_slot, :, :]
                out_ref[pl.ds(c_l * chunk + half, half), :] = comm_l[r_slot, :, :]

            if g < n_steps - 1:
                pl.semaphore_signal(credit_r, inc=1, device_id=(left,),
                                    device_id_type=pl.DeviceIdType.MESH)
                pl.semaphore_signal(credit_l, inc=1, device_id=(right,),
                                    device_id_type=pl.DeviceIdType.MESH)

    return pl.pallas_call(
        body,
        out_shape=jax.ShapeDtypeStruct((m, n), jnp.float32),
        in_specs=[
            pl.BlockSpec(memory_space=pltpu.VMEM),
            pl.BlockSpec(memory_space=pltpu.VMEM),
        ],
        out_specs=pl.BlockSpec(memory_space=pltpu.VMEM),
        scratch_shapes=[
            pltpu.VMEM((2, half, n), jnp.float32),
            pltpu.VMEM((2, half, n), jnp.float32),
            pltpu.SemaphoreType.DMA((2,)),
            pltpu.SemaphoreType.DMA((2,)),
            pltpu.SemaphoreType.DMA((2,)),
            pltpu.SemaphoreType.DMA((2,)),
            pltpu.SemaphoreType.REGULAR,
            pltpu.SemaphoreType.REGULAR,
        ],
        compiler_params=pltpu.CompilerParams(collective_id=0),
    )(A, B)


# device time: 215305 ns/iter; 2.4834x vs baseline; 1.6660x over previous
import jax
import jax.numpy as jnp
from jax import lax
from jax.experimental import pallas as pl
from jax.experimental.pallas import tpu as pltpu

N_DEV = 16


def kernel(A, B):
    m, k = A.shape
    _, n = B.shape
    chunk = m // N_DEV

    def body(a_ref, b_ref, out_ref, zbf_ref, rs_ref,
             rs_recv_sems, ag_recv_sems, rs_send_sems, ag_send_sems):
        my = lax.axis_index("i")

        def rows(c):
            return pl.ds(c * chunk, chunk)

        def partial(c):
            a_blk = a_ref[rows(c), :]
            return jnp.dot(a_blk, b_ref[...], preferred_element_type=jnp.float32)

        barrier_sem = pltpu.get_barrier_semaphore()
        for d in range(1, N_DEV):
            pl.semaphore_signal(
                barrier_sem, inc=1,
                device_id=(jnp.mod(my + d, N_DEV),),
                device_id_type=pl.DeviceIdType.MESH,
            )
        pl.semaphore_wait(barrier_sem, N_DEV - 1)

        rs_descs = []
        for d in range(1, N_DEV):
            tgt = jnp.mod(my + d, N_DEV)
            zbf_ref[rows(tgt), :] = partial(tgt).astype(jnp.bfloat16)
            desc = pltpu.make_async_remote_copy(
                src_ref=zbf_ref.at[rows(tgt), :],
                dst_ref=rs_ref.at[rows(my), :],
                send_sem=rs_send_sems.at[d],
                recv_sem=rs_recv_sems.at[my],
                device_id=(tgt,),
                device_id_type=pl.DeviceIdType.MESH,
            )
            desc.start()
            rs_descs.append(desc)

        acc = partial(my)

        for d in range(1, N_DEV):
            src = jnp.mod(my + d, N_DEV)
            recv = pltpu.make_async_remote_copy(
                src_ref=rs_ref.at[rows(src), :],
                dst_ref=rs_ref.at[rows(src), :],
                send_sem=rs_send_sems.at[0],
                recv_sem=rs_recv_sems.at[src],
                device_id=(src,),
                device_id_type=pl.DeviceIdType.MESH,
            )
            recv.wait_recv()
            acc = acc + rs_ref[rows(src), :].astype(jnp.float32)

        z = acc / (1.0 + jnp.exp(-acc))
        out_ref[rows(my), :] = z.astype(jnp.bfloat16)

        ag_descs = []
        for d in range(1, N_DEV):
            tgt = jnp.mod(my + d, N_DEV)
            desc = pltpu.make_async_remote_copy(
                src_ref=out_ref.at[rows(my), :],
                dst_ref=out_ref.at[rows(my), :],
                send_sem=ag_send_sems.at[d],
                recv_sem=ag_recv_sems.at[my],
                device_id=(tgt,),
                device_id_type=pl.DeviceIdType.MESH,
            )
            desc.start()
            ag_descs.append(desc)

        for d in range(1, N_DEV):
            src = jnp.mod(my + d, N_DEV)
            recv = pltpu.make_async_remote_copy(
                src_ref=out_ref.at[rows(src), :],
                dst_ref=out_ref.at[rows(src), :],
                send_sem=ag_send_sems.at[0],
                recv_sem=ag_recv_sems.at[src],
                device_id=(src,),
                device_id_type=pl.DeviceIdType.MESH,
            )
            recv.wait_recv()

        for desc in rs_descs + ag_descs:
            desc.wait_send()

    return pl.pallas_call(
        body,
        out_shape=jax.ShapeDtypeStruct((m, n), jnp.bfloat16),
        in_specs=[
            pl.BlockSpec(memory_space=pltpu.VMEM),
            pl.BlockSpec(memory_space=pltpu.VMEM),
        ],
        out_specs=pl.BlockSpec(memory_space=pltpu.VMEM),
        scratch_shapes=[
            pltpu.VMEM((m, n), jnp.bfloat16),
            pltpu.VMEM((m, n), jnp.bfloat16),
            pltpu.SemaphoreType.DMA((N_DEV,)),
            pltpu.SemaphoreType.DMA((N_DEV,)),
            pltpu.SemaphoreType.DMA((N_DEV,)),
            pltpu.SemaphoreType.DMA((N_DEV,)),
        ],
        compiler_params=pltpu.CompilerParams(
            collective_id=0,
            vmem_limit_bytes=45 * 1024 * 1024,
        ),
    )(A, B)


# device time: 137369 ns/iter; 3.8923x vs baseline; 1.5673x over previous
import jax
import jax.numpy as jnp
from jax import lax
from jax.experimental import pallas as pl
from jax.experimental.pallas import tpu as pltpu

N_DEV = 16


def kernel(A, B):
    m, k = A.shape
    _, n = B.shape
    grp = m // 4
    half = grp // 2
    chunk = m // N_DEV

    f32 = jnp.float32
    bf16 = jnp.bfloat16
    MESH = pl.DeviceIdType.MESH

    def body(a_ref, b_ref, out_ref, comm_r, comm_l, zred_ref, zrs_ref,
             r_send, r_recv, l_send, l_recv,
             zrs_send, zrs_recv, zag_send, zag_recv,
             ag_send, ag_recv):
        my = lax.axis_index("i")
        p = jnp.mod(my, 4)
        zpos = my // 4
        base = my - p

        def pr_id(q):
            return base + jnp.mod(q, 4)

        def col_id(zq):
            return jnp.mod(zq, 4) * 4 + p

        right = pr_id(p + 1)
        left = pr_id(p - 1)

        def top_rows(g):
            return pl.ds(jnp.mod(g, 4) * grp, half)

        def bot_rows(g):
            return pl.ds(jnp.mod(g, 4) * grp + half, half)

        def partial(row_ds):
            return jnp.dot(a_ref[row_ds, :], b_ref[...],
                           preferred_element_type=f32)

        def copy(src, dst, send, recv, dev):
            return pltpu.make_async_remote_copy(
                src_ref=src, dst_ref=dst, send_sem=send, recv_sem=recv,
                device_id=(dev,), device_id_type=MESH)

        partners = (left, right, pr_id(p + 2),
                    col_id(zpos + 1), col_id(zpos + 2), col_id(zpos + 3))
        barrier = pltpu.get_barrier_semaphore()
        for tgt in partners:
            pl.semaphore_signal(barrier, inc=1, device_id=(tgt,),
                                device_id_type=MESH)
        pl.semaphore_wait(barrier, 6)

        comm_r[0, :, :] = partial(top_rows(p + 3)).astype(bf16)
        comm_l[0, :, :] = partial(bot_rows(p + 1)).astype(bf16)

        for s in range(3):
            rr = copy(comm_r.at[s], comm_r.at[s + 1],
                      r_send.at[s], r_recv.at[s], right)
            ll = copy(comm_l.at[s], comm_l.at[s + 1],
                      l_send.at[s], l_recv.at[s], left)
            rr.start()
            ll.start()
            p_r = partial(top_rows(p + 2 - s))
            p_l = partial(bot_rows(p + 2 + s))
            rr.wait()
            ll.wait()
            if s < 2:
                comm_r[s + 1, :, :] = (
                    comm_r[s + 1, :, :].astype(f32) + p_r).astype(bf16)
                comm_l[s + 1, :, :] = (
                    comm_l[s + 1, :, :].astype(f32) + p_l).astype(bf16)
            else:
                zred_ref[pl.ds(0, half), :] = (
                    comm_r[3, :, :].astype(f32) + p_r).astype(bf16)
                zred_ref[pl.ds(half, half), :] = (
                    comm_l[3, :, :].astype(f32) + p_l).astype(bf16)

        send_descs = []
        for dz in range(1, 4):
            zq = jnp.mod(zpos + dz, 4)
            d = copy(zred_ref.at[pl.ds(zq * chunk, chunk), :],
                     zrs_ref.at[pl.ds(zpos * chunk, chunk), :],
                     zrs_send.at[dz - 1], zrs_recv.at[zpos], col_id(zpos + dz))
            d.start()
            send_descs.append(d)

        acc = zred_ref[pl.ds(zpos * chunk, chunk), :].astype(f32)
        for dz in range(1, 4):
            zq = jnp.mod(zpos + dz, 4)
            slot = zrs_ref.at[pl.ds(zq * chunk, chunk), :]
            copy(slot, slot, zrs_send.at[0], zrs_recv.at[zq], my).wait_recv()
            acc = acc + zrs_ref[pl.ds(zq * chunk, chunk), :].astype(f32)

        zs = acc / (1.0 + jnp.exp(-acc))
        my_rows = pl.ds(p * grp + zpos * chunk, chunk)
        out_ref[my_rows, :] = zs.astype(bf16)

        for dz in range(1, 4):
            d = copy(out_ref.at[my_rows, :], out_ref.at[my_rows, :],
                     zag_send.at[dz - 1], zag_recv.at[zpos], col_id(zpos + dz))
            d.start()
            send_descs.append(d)
        for dz in range(1, 4):
            zq = jnp.mod(zpos + dz, 4)
            slot = out_ref.at[pl.ds(p * grp + zq * chunk, chunk), :]
            copy(slot, slot, zag_send.at[0], zag_recv.at[zq], my).wait_recv()

        for s in range(3):
            sr = copy(out_ref.at[top_rows(p - s), :],
                      out_ref.at[top_rows(p - s), :],
                      ag_send.at[s], ag_recv.at[s], right)
            sl = copy(out_ref.at[bot_rows(p + s), :],
                      out_ref.at[bot_rows(p + s), :],
                      ag_send.at[3 + s], ag_recv.at[3 + s], left)
            sr.start()
            sl.start()
            send_descs.append(sr)
            send_descs.append(sl)
            slot_r = out_ref.at[top_rows(p - s - 1), :]
            copy(slot_r, slot_r, ag_send.at[0], ag_recv.at[s], my).wait_recv()
            slot_l = out_ref.at[bot_rows(p + s + 1), :]
            copy(slot_l, slot_l, ag_send.at[0], ag_recv.at[3 + s],
                 my).wait_recv()

        for d in send_descs:
            d.wait_send()

    return pl.pallas_call(
        body,
        out_shape=jax.ShapeDtypeStruct((m, n), bf16),
        in_specs=[
            pl.BlockSpec(memory_space=pltpu.VMEM),
            pl.BlockSpec(memory_space=pltpu.VMEM),
        ],
        out_specs=pl.BlockSpec(memory_space=pltpu.VMEM),
        scratch_shapes=[
            pltpu.VMEM((4, half, n), bf16),
            pltpu.VMEM((4, half, n), bf16),
            pltpu.VMEM((grp, n), bf16),
            pltpu.VMEM((grp, n), bf16),
            pltpu.SemaphoreType.DMA((3,)),
            pltpu.SemaphoreType.DMA((3,)),
            pltpu.SemaphoreType.DMA((3,)),
            pltpu.SemaphoreType.DMA((3,)),
            pltpu.SemaphoreType.DMA((3,)),
            pltpu.SemaphoreType.DMA((4,)),
            pltpu.SemaphoreType.DMA((3,)),
            pltpu.SemaphoreType.DMA((4,)),
            pltpu.SemaphoreType.DMA((6,)),
            pltpu.SemaphoreType.DMA((6,)),
        ],
        compiler_params=pltpu.CompilerParams(
            collective_id=0,
            vmem_limit_bytes=45 * 1024 * 1024,
        ),
    )(A, B)


# device time: 128134 ns/iter; 4.1729x vs baseline; 1.0721x over previous
import jax
import jax.numpy as jnp
from jax import lax
from jax.experimental import pallas as pl
from jax.experimental.pallas import tpu as pltpu

N_DEV = 16


def kernel(A, B):
    m, k = A.shape
    _, n = B.shape
    grp = m // 4
    half = grp // 2
    chunk = m // N_DEV
    nh = n // 2

    f32 = jnp.float32
    bf16 = jnp.bfloat16
    MESH = pl.DeviceIdType.MESH

    def body(a_ref, b_ref, out_ref, comm_r, comm_l, zred_ref, zrs_ref,
             r_send, r_recv, l_send, l_recv,
             zrs_send, zrs_recv, zag_send, zag_recv,
             ag_send, ag_recv):
        my = lax.axis_index("i")
        p = jnp.mod(my, 4)
        zpos = my // 4
        base = my - p

        def pr_id(q):
            return base + jnp.mod(q, 4)

        def col_id(zq):
            return jnp.mod(zq, 4) * 4 + p

        right = pr_id(p + 1)
        left = pr_id(p - 1)
        my_rows = pl.ds(p * grp + zpos * chunk, chunk)

        def cols(h):
            return pl.ds(h * nh, nh)

        def top_rows(g):
            return pl.ds(jnp.mod(g, 4) * grp, half)

        def bot_rows(g):
            return pl.ds(jnp.mod(g, 4) * grp + half, half)

        def partial(row_ds, h):
            return jnp.dot(a_ref[row_ds, :], b_ref[:, cols(h)],
                           preferred_element_type=f32)

        def copy(src, dst, send, recv, dev):
            return pltpu.make_async_remote_copy(
                src_ref=src, dst_ref=dst, send_sem=send, recv_sem=recv,
                device_id=(dev,), device_id_type=MESH)

        send_descs = []


        def p1_stage(h):
            comm_r[h, 0, :, :] = partial(top_rows(p + 3), h).astype(bf16)
            comm_l[h, 0, :, :] = partial(bot_rows(p + 1), h).astype(bf16)

        def p1_start(h, s):
            rr = copy(comm_r.at[h, s], comm_r.at[h, s + 1],
                      r_send.at[h * 3 + s], r_recv.at[h * 3 + s], right)
            ll = copy(comm_l.at[h, s], comm_l.at[h, s + 1],
                      l_send.at[h * 3 + s], l_recv.at[h * 3 + s], left)
            rr.start()
            ll.start()
            return rr, ll

        def p1_finish(h, s, rr, ll):
            p_r = partial(top_rows(p + 2 - s), h)
            p_l = partial(bot_rows(p + 2 + s), h)
            rr.wait()
            ll.wait()
            if s < 2:
                comm_r[h, s + 1, :, :] = (
                    comm_r[h, s + 1, :, :].astype(f32) + p_r).astype(bf16)
                comm_l[h, s + 1, :, :] = (
                    comm_l[h, s + 1, :, :].astype(f32) + p_l).astype(bf16)
            else:
                zred_ref[h, pl.ds(0, half), :] = (
                    comm_r[h, 3, :, :].astype(f32) + p_r).astype(bf16)
                zred_ref[h, pl.ds(half, half), :] = (
                    comm_l[h, 3, :, :].astype(f32) + p_l).astype(bf16)

        def p2_send(h):
            for dz in range(1, 4):
                zq = jnp.mod(zpos + dz, 4)
                d = copy(zred_ref.at[h, pl.ds(zq * chunk, chunk), :],
                         zrs_ref.at[h, pl.ds(zpos * chunk, chunk), :],
                         zrs_send.at[h * 3 + dz - 1],
                         zrs_recv.at[h * 4 + zpos], col_id(zpos + dz))
                d.start()
                send_descs.append(d)

        def p2_reduce_silu(h):
            acc = zred_ref[h, pl.ds(zpos * chunk, chunk), :].astype(f32)
            for dz in range(1, 4):
                zq = jnp.mod(zpos + dz, 4)
                slot = zrs_ref.at[h, pl.ds(zq * chunk, chunk), :]
                copy(slot, slot, zrs_send.at[0],
                     zrs_recv.at[h * 4 + zq], my).wait_recv()
                acc = acc + zrs_ref[h, pl.ds(zq * chunk, chunk), :].astype(f32)
            zs = acc / (1.0 + jnp.exp(-acc))
            out_ref[my_rows, cols(h)] = zs.astype(bf16)

        def p3_send(h):
            for dz in range(1, 4):
                d = copy(out_ref.at[my_rows, cols(h)],
                         out_ref.at[my_rows, cols(h)],
                         zag_send.at[h * 3 + dz - 1],
                         zag_recv.at[h * 4 + zpos], col_id(zpos + dz))
                d.start()
                send_descs.append(d)

        def p3_wait(h):
            for dz in range(1, 4):
                zq = jnp.mod(zpos + dz, 4)
                slot = out_ref.at[pl.ds(p * grp + zq * chunk, chunk), cols(h)]
                copy(slot, slot, zag_send.at[0],
                     zag_recv.at[h * 4 + zq], my).wait_recv()

        def p4_start(h, s):
            sr = copy(out_ref.at[top_rows(p - s), cols(h)],
                      out_ref.at[top_rows(p - s), cols(h)],
                      ag_send.at[h * 6 + s], ag_recv.at[h * 6 + s], right)
            sl = copy(out_ref.at[bot_rows(p + s), cols(h)],
                      out_ref.at[bot_rows(p + s), cols(h)],
                      ag_send.at[h * 6 + 3 + s], ag_recv.at[h * 6 + 3 + s],
                      left)
            sr.start()
            sl.start()
            send_descs.append(sr)
            send_descs.append(sl)

        def p4_finish(h, s):
            slot_r = out_ref.at[top_rows(p - s - 1), cols(h)]
            copy(slot_r, slot_r, ag_send.at[0],
                 ag_recv.at[h * 6 + s], my).wait_recv()
            slot_l = out_ref.at[bot_rows(p + s + 1), cols(h)]
            copy(slot_l, slot_l, ag_send.at[0],
                 ag_recv.at[h * 6 + 3 + s], my).wait_recv()

        partners = (left, right, pr_id(p + 2),
                    col_id(zpos + 1), col_id(zpos + 2), col_id(zpos + 3))
        barrier = pltpu.get_barrier_semaphore()
        for tgt in partners:
            pl.semaphore_signal(barrier, inc=1, device_id=(tgt,),
                                device_id_type=MESH)
        pl.semaphore_wait(barrier, 6)

        p1_stage(0)
        h0 = p1_start(0, 0)
        p1_stage(1)
        p1_finish(0, 0, *h0)
        h1 = p1_start(1, 0)
        h0 = p1_start(0, 1)
        p1_finish(1, 0, *h1)
        p1_finish(0, 1, *h0)
        h1 = p1_start(1, 1)
        h0 = p1_start(0, 2)
        p1_finish(1, 1, *h1)
        p1_finish(0, 2, *h0)
        p2_send(0)
        h1 = p1_start(1, 2)
        p2_reduce_silu(0)
        p3_send(0)
        p1_finish(1, 2, *h1)
        p2_send(1)
        p3_wait(0)
        p2_reduce_silu(1)
        p3_send(1)
        p4_start(0, 0)
        p4_finish(0, 0)
        p3_wait(1)
        p4_start(1, 0)
        p4_start(0, 1)
        p4_finish(1, 0)
        p4_finish(0, 1)
        p4_start(1, 1)
        p4_start(0, 2)
        p4_finish(1, 1)
        p4_finish(0, 2)
        p4_start(1, 2)
        p4_finish(1, 2)

        for d in send_descs:
            d.wait_send()

    return pl.pallas_call(
        body,
        out_shape=jax.ShapeDtypeStruct((m, n), bf16),
        in_specs=[
            pl.BlockSpec(memory_space=pltpu.VMEM),
            pl.BlockSpec(memory_space=pltpu.VMEM),
        ],
        out_specs=pl.BlockSpec(memory_space=pltpu.VMEM),
        scratch_shapes=[
            pltpu.VMEM((2, 4, half, nh), bf16),
            pltpu.VMEM((2, 4, half, nh), bf16),
            pltpu.VMEM((2, grp, nh), bf16),
            pltpu.VMEM((2, grp, nh), bf16),
            pltpu.SemaphoreType.DMA((6,)),
            pltpu.SemaphoreType.DMA((6,)),
            pltpu.SemaphoreType.DMA((6,)),
            pltpu.SemaphoreType.DMA((6,)),
            pltpu.SemaphoreType.DMA((6,)),
            pltpu.SemaphoreType.DMA((8,)),
            pltpu.SemaphoreType.DMA((6,)),
            pltpu.SemaphoreType.DMA((8,)),
            pltpu.SemaphoreType.DMA((12,)),
            pltpu.SemaphoreType.DMA((12,)),
        ],
        compiler_params=pltpu.CompilerParams(
            collective_id=0,
            vmem_limit_bytes=45 * 1024 * 1024,
        ),
    )(A, B)


# device time: 98183 ns/iter; 5.4458x vs baseline; 1.3051x over previous
import jax
import jax.numpy as jnp
from jax import lax
from jax.experimental import pallas as pl
from jax.experimental.pallas import tpu as pltpu

N_DEV = 16


def kernel(A, B):
    m, k = A.shape
    _, n = B.shape
    grp = m // 4
    half = grp // 2
    chunk = m // N_DEV
    nh = n // 4

    f32 = jnp.float32
    bf16 = jnp.bfloat16
    MESH = pl.DeviceIdType.MESH

    def body(a_ref, b_ref, out_ref, comm_r, comm_l, zred_ref, zrs_ref,
             r_send, r_recv, l_send, l_recv,
             zrs_send, zrs_recv, zag_send, zag_recv,
             ag_send, ag_recv):
        my = lax.axis_index("i")
        p = jnp.mod(my, 4)
        zpos = my // 4
        base = my - p

        def pr_id(q):
            return base + jnp.mod(q, 4)

        def col_id(zq):
            return jnp.mod(zq, 4) * 4 + p

        right = pr_id(p + 1)
        left = pr_id(p - 1)
        my_rows = pl.ds(p * grp + zpos * chunk, chunk)

        def cols(h):
            return pl.ds(h * nh, nh)

        def top_rows(g):
            return pl.ds(jnp.mod(g, 4) * grp, half)

        def bot_rows(g):
            return pl.ds(jnp.mod(g, 4) * grp + half, half)

        def partial(row_ds, h):
            return jnp.dot(a_ref[row_ds, :], b_ref[:, cols(h)],
                           preferred_element_type=f32)

        def copy(src, dst, send, recv, dev):
            return pltpu.make_async_remote_copy(
                src_ref=src, dst_ref=dst, send_sem=send, recv_sem=recv,
                device_id=(dev,), device_id_type=MESH)

        send_descs = []


        def p1_stage(h):
            comm_r[h, 0, :, :] = partial(top_rows(p + 3), h).astype(bf16)
            comm_l[h, 0, :, :] = partial(bot_rows(p + 1), h).astype(bf16)

        def p1_start(h, s):
            rr = copy(comm_r.at[h, s], comm_r.at[h, s + 1],
                      r_send.at[h * 3 + s], r_recv.at[h * 3 + s], right)
            ll = copy(comm_l.at[h, s], comm_l.at[h, s + 1],
                      l_send.at[h * 3 + s], l_recv.at[h * 3 + s], left)
            rr.start()
            ll.start()
            return rr, ll

        def p1_finish(h, s, rr, ll):
            p_r = partial(top_rows(p + 2 - s), h)
            p_l = partial(bot_rows(p + 2 + s), h)
            rr.wait()
            ll.wait()
            if s < 2:
                comm_r[h, s + 1, :, :] = (
                    comm_r[h, s + 1, :, :].astype(f32) + p_r).astype(bf16)
                comm_l[h, s + 1, :, :] = (
                    comm_l[h, s + 1, :, :].astype(f32) + p_l).astype(bf16)
            else:
                zred_ref[h, pl.ds(0, half), :] = (
                    comm_r[h, 3, :, :].astype(f32) + p_r).astype(bf16)
                zred_ref[h, pl.ds(half, half), :] = (
                    comm_l[h, 3, :, :].astype(f32) + p_l).astype(bf16)

        def p2_send(h):
            for dz in range(1, 4):
                zq = jnp.mod(zpos + dz, 4)
                d = copy(zred_ref.at[h, pl.ds(zq * chunk, chunk), :],
                         zrs_ref.at[h, pl.ds(zpos * chunk, chunk), :],
                         zrs_send.at[h * 3 + dz - 1],
                         zrs_recv.at[h * 4 + zpos], col_id(zpos + dz))
                d.start()
                send_descs.append(d)

        def p2_reduce_silu(h):
            acc = zred_ref[h, pl.ds(zpos * chunk, chunk), :].astype(f32)
            for dz in range(1, 4):
                zq = jnp.mod(zpos + dz, 4)
                slot = zrs_ref.at[h, pl.ds(zq * chunk, chunk), :]
                copy(slot, slot, zrs_send.at[0],
                     zrs_recv.at[h * 4 + zq], my).wait_recv()
                acc = acc + zrs_ref[h, pl.ds(zq * chunk, chunk), :].astype(f32)
            zs = acc / (1.0 + jnp.exp(-acc))
            out_ref[my_rows, cols(h)] = zs.astype(bf16)

        def p3_send(h):
            for dz in range(1, 4):
                d = copy(out_ref.at[my_rows, cols(h)],
                         out_ref.at[my_rows, cols(h)],
                         zag_send.at[h * 3 + dz - 1],
                         zag_recv.at[h * 4 + zpos], col_id(zpos + dz))
                d.start()
                send_descs.append(d)

        def p3_wait(h):
            for dz in range(1, 4):
                zq = jnp.mod(zpos + dz, 4)
                slot = out_ref.at[pl.ds(p * grp + zq * chunk, chunk), cols(h)]
                copy(slot, slot, zag_send.at[0],
                     zag_recv.at[h * 4 + zq], my).wait_recv()

        def p4_start(h, s):
            sr = copy(out_ref.at[top_rows(p - s), cols(h)],
                      out_ref.at[top_rows(p - s), cols(h)],
                      ag_send.at[h * 6 + s], ag_recv.at[h * 6 + s], right)
            sl = copy(out_ref.at[bot_rows(p + s), cols(h)],
                      out_ref.at[bot_rows(p + s), cols(h)],
                      ag_send.at[h * 6 + 3 + s], ag_recv.at[h * 6 + 3 + s],
                      left)
            sr.start()
            sl.start()
            send_descs.append(sr)
            send_descs.append(sl)

        def p4_finish(h, s):
            slot_r = out_ref.at[top_rows(p - s - 1), cols(h)]
            copy(slot_r, slot_r, ag_send.at[0],
                 ag_recv.at[h * 6 + s], my).wait_recv()
            slot_l = out_ref.at[bot_rows(p + s + 1), cols(h)]
            copy(slot_l, slot_l, ag_send.at[0],
                 ag_recv.at[h * 6 + 3 + s], my).wait_recv()

        partners = (left, right, pr_id(p + 2),
                    col_id(zpos + 1), col_id(zpos + 2), col_id(zpos + 3))
        barrier = pltpu.get_barrier_semaphore()
        for tgt in partners:
            pl.semaphore_signal(barrier, inc=1, device_id=(tgt,),
                                device_id_type=MESH)
        pl.semaphore_wait(barrier, 6)

        d = {}
        p1_stage(0)
        d[0] = p1_start(0, 0)
        p1_stage(1)
        p1_finish(0, 0, *d[0])
        d[0] = p1_start(0, 1)
        d[1] = p1_start(1, 0)
        p1_finish(0, 1, *d[0])
        d[0] = p1_start(0, 2)
        p1_stage(2)
        p1_finish(1, 0, *d[1])
        d[1] = p1_start(1, 1)
        p1_finish(0, 2, *d[0])
        p2_send(0)
        d[2] = p1_start(2, 0)
        p1_finish(1, 1, *d[1])
        d[1] = p1_start(1, 2)
        p2_reduce_silu(0)
        p3_send(0)
        p1_stage(3)
        p1_finish(2, 0, *d[2])
        d[2] = p1_start(2, 1)
        p1_finish(1, 2, *d[1])
        p2_send(1)
        d[3] = p1_start(3, 0)
        p3_wait(0)
        p1_finish(2, 1, *d[2])
        d[2] = p1_start(2, 2)
        p2_reduce_silu(1)
        p3_send(1)
        p4_start(0, 0)
        p1_finish(3, 0, *d[3])
        d[3] = p1_start(3, 1)
        p1_finish(2, 2, *d[2])
        p2_send(2)
        p4_finish(0, 0)
        p4_start(0, 1)
        p3_wait(1)
        p1_finish(3, 1, *d[3])
        d[3] = p1_start(3, 2)
        p2_reduce_silu(2)
        p3_send(2)
        p4_start(1, 0)
        p4_finish(0, 1)
        p4_start(0, 2)
        p1_finish(3, 2, *d[3])
        p2_send(3)
        p4_finish(1, 0)
        p4_start(1, 1)
        p3_wait(2)
        p2_reduce_silu(3)
        p3_send(3)
        p4_start(2, 0)
        p4_finish(0, 2)
        p4_finish(1, 1)
        p4_start(1, 2)
        p3_wait(3)
        p4_finish(2, 0)
        p4_start(2, 1)
        p4_start(3, 0)
        p4_finish(1, 2)
        p4_finish(2, 1)
        p4_start(2, 2)
        p4_finish(3, 0)
        p4_start(3, 1)
        p4_finish(2, 2)
        p4_finish(3, 1)
        p4_start(3, 2)
        p4_finish(3, 2)

        for d in send_descs:
            d.wait_send()

    return pl.pallas_call(
        body,
        out_shape=jax.ShapeDtypeStruct((m, n), bf16),
        in_specs=[
            pl.BlockSpec(memory_space=pltpu.VMEM),
            pl.BlockSpec(memory_space=pltpu.VMEM),
        ],
        out_specs=pl.BlockSpec(memory_space=pltpu.VMEM),
        scratch_shapes=[
            pltpu.VMEM((4, 4, half, nh), bf16),
            pltpu.VMEM((4, 4, half, nh), bf16),
            pltpu.VMEM((4, grp, nh), bf16),
            pltpu.VMEM((4, grp, nh), bf16),
            pltpu.SemaphoreType.DMA((12,)),
            pltpu.SemaphoreType.DMA((12,)),
            pltpu.SemaphoreType.DMA((12,)),
            pltpu.SemaphoreType.DMA((12,)),
            pltpu.SemaphoreType.DMA((12,)),
            pltpu.SemaphoreType.DMA((16,)),
            pltpu.SemaphoreType.DMA((12,)),
            pltpu.SemaphoreType.DMA((16,)),
            pltpu.SemaphoreType.DMA((24,)),
            pltpu.SemaphoreType.DMA((24,)),
        ],
        compiler_params=pltpu.CompilerParams(
            collective_id=0,
            vmem_limit_bytes=45 * 1024 * 1024,
        ),
    )(A, B)
